# baseline (speedup 1.0000x reference)
import functools

import jax
import jax.numpy as jnp
from jax import lax
from jax.experimental import pallas as pl
from jax.experimental.pallas import tpu as pltpu
from jax.experimental.pallas import tpu_sc as plsc

EMBEDDING_DIM = 64
NUM_CORES = 2
NUM_SUBCORES = 16
NUM_WORKERS = NUM_CORES * NUM_SUBCORES
CHUNK = 128

_mesh = plsc.VectorSubcoreMesh(core_axis_name="c", subcore_axis_name="s")


def _make_lookup(batch, dim):
    per_worker = batch // NUM_WORKERS
    n_chunks = per_worker // CHUNK

    @functools.partial(
        pl.kernel,
        mesh=_mesh,
        out_type=jax.ShapeDtypeStruct((batch, dim), jnp.float32),
        scratch_types=[
            pltpu.VMEM((CHUNK,), jnp.int32),
            pltpu.VMEM((CHUNK, dim), jnp.float32),
            pltpu.SemaphoreType.DMA,
        ],
        compiler_params=pltpu.CompilerParams(use_tc_tiling_on_sc=False),
    )
    def lookup(idx_hbm, table_hbm, out_hbm, idx_v, rows_v, sem):
        wid = lax.axis_index("s") * NUM_CORES + lax.axis_index("c")
        base = wid * per_worker

        def body(j, carry):
            off = base + j * CHUNK
            pltpu.sync_copy(idx_hbm.at[pl.ds(off, CHUNK)], idx_v)
            pltpu.async_copy(table_hbm.at[idx_v], rows_v, sem).wait()
            pltpu.sync_copy(rows_v, out_hbm.at[pl.ds(off, CHUNK)])
            return carry

        lax.fori_loop(0, n_chunks, body, 0)

    return lookup


def kernel(token_ids, embedding_matrix):
    b, s = token_ids.shape
    dim = embedding_matrix.shape[1]
    flat = token_ids.reshape(-1).astype(jnp.int32)
    out = _make_lookup(flat.shape[0], dim)(flat, embedding_matrix)
    return out.reshape(b, s, dim)


# speedup vs baseline: 1.1898x; 1.1898x over previous
import functools

import jax
import jax.numpy as jnp
from jax import lax
from jax.experimental import pallas as pl
from jax.experimental.pallas import tpu as pltpu
from jax.experimental.pallas import tpu_sc as plsc

NUM_CORES = 2
NUM_SUBCORES = 16
NUM_WORKERS = NUM_CORES * NUM_SUBCORES
CHUNK = 128
GROUP = 4
GCHUNK = CHUNK * GROUP

_mesh = plsc.VectorSubcoreMesh(core_axis_name="c", subcore_axis_name="s")


def _make_lookup(batch, dim):
    per_worker = batch // NUM_WORKERS
    n_groups = per_worker // GCHUNK
    n_pairs = n_groups // 2

    @functools.partial(
        pl.kernel,
        mesh=_mesh,
        out_type=jax.ShapeDtypeStruct((batch, dim), jnp.float32),
        scratch_types=[
            pltpu.VMEM((per_worker,), jnp.int32),
            pltpu.VMEM((GCHUNK, dim), jnp.float32),
            pltpu.VMEM((GCHUNK, dim), jnp.float32),
            pltpu.SemaphoreType.DMA,
            pltpu.SemaphoreType.DMA,
            pltpu.SemaphoreType.DMA,
            pltpu.SemaphoreType.DMA,
        ],
        compiler_params=pltpu.CompilerParams(use_tc_tiling_on_sc=False),
    )
    def lookup(idx_hbm, table_hbm, out_hbm, idx_v, rows0, rows1,
               gsem0, gsem1, ssem0, ssem1):
        wid = lax.axis_index("s") * NUM_CORES + lax.axis_index("c")
        base = wid * per_worker
        pltpu.sync_copy(idx_hbm.at[pl.ds(base, per_worker)], idx_v)

        def fire(g, rows, gsem):
            for b in range(GROUP):
                off = g * GCHUNK + b * CHUNK
                pltpu.async_copy(
                    table_hbm.at[idx_v.at[pl.ds(off, CHUNK)]],
                    rows.at[pl.ds(b * CHUNK, CHUNK)],
                    gsem,
                )

        def drain_gathers(rows, gsem):
            pltpu.make_async_copy(
                table_hbm.at[pl.ds(0, GCHUNK)], rows, gsem).wait()

        def start_store(g, rows, ssem):
            pltpu.async_copy(
                rows, out_hbm.at[pl.ds(base + g * GCHUNK, GCHUNK)], ssem)

        def wait_store(g, rows, ssem):
            pltpu.make_async_copy(
                rows, out_hbm.at[pl.ds(base + g * GCHUNK, GCHUNK)], ssem).wait()

        fire(0, rows0, gsem0)
        fire(1, rows1, gsem1)

        def body(p, carry):
            g0 = 2 * p
            g1 = 2 * p + 1
            drain_gathers(rows0, gsem0)
            start_store(g0, rows0, ssem0)
            drain_gathers(rows1, gsem1)
            start_store(g1, rows1, ssem1)

            @pl.when(p + 1 < n_pairs)
            def _():
                wait_store(g0, rows0, ssem0)
                fire(g0 + 2, rows0, gsem0)
                wait_store(g1, rows1, ssem1)
                fire(g1 + 2, rows1, gsem1)

            return carry

        lax.fori_loop(0, n_pairs, body, 0)
        wait_store(n_groups - 2, rows0, ssem0)
        wait_store(n_groups - 1, rows1, ssem1)

    return lookup


def kernel(token_ids, embedding_matrix):
    b, s = token_ids.shape
    dim = embedding_matrix.shape[1]
    flat = token_ids.reshape(-1).astype(jnp.int32)
    out = _make_lookup(flat.shape[0], dim)(flat, embedding_matrix)
    return out.reshape(b, s, dim)
